# Initial kernel scaffold; baseline (speedup 1.0000x reference)
import jax
import jax.numpy as jnp
from jax.experimental import pallas as pl


def kernel(kemb, pos, batch_index, W1, b1, W2, b2, W_out):
    raise NotImplementedError("write your pallas kernel here")


# speedup vs baseline: 6.9743x; 6.9743x over previous
import functools

import jax
import jax.numpy as jnp
from jax.experimental import pallas as pl
from jax.experimental.pallas import tpu as pltpu

N = 100000
H = 128
G = 512
BLK = 1024


def _fused_kernel(kemb_ref, a_ref, b_ref, bidx_ref,
                  W1_ref, b1_ref, W2_ref, b2_ref, Wout_ref,
                  out_ref, acc_ref, *, nblocks):
    i = pl.program_id(0)

    @pl.when(i == 0)
    def _init():
        acc_ref[...] = jnp.zeros_like(acc_ref)

    x = kemb_ref[...]
    h = jax.nn.silu(jnp.dot(x, W1_ref[...], preferred_element_type=jnp.float32)
                    + b1_ref[...])
    h = jax.nn.silu(jnp.dot(h, W2_ref[...], preferred_element_type=jnp.float32)
                    + b2_ref[...])
    q = jnp.dot(x + h, Wout_ref[...], preferred_element_type=jnp.float32)

    data = q * a_ref[...] + b_ref[...]

    bidx = bidx_ref[0]
    onehot = (jax.lax.broadcasted_iota(jnp.int32, (G, BLK), 0) == bidx
              ).astype(jnp.float32)
    acc_ref[...] += jnp.dot(onehot, data, preferred_element_type=jnp.float32)

    @pl.when(i == nblocks - 1)
    def _final():
        acc = acc_ref[...]
        sqp = acc[:, 0:3]
        sp = acc[:, 3:6]
        sq = acc[:, 6:7]
        n = acc[:, 7:8]
        ctr = sqp - sq * (sp / jnp.maximum(n, 1.0))
        out_ref[...] = jnp.sqrt(jnp.sum(ctr * ctr, axis=1, keepdims=True))


def kernel(kemb, pos, batch_index, W1, b1, W2, b2, W_out):
    nblocks = (N + BLK - 1) // BLK
    npad = nblocks * BLK

    kemb_p = jnp.pad(kemb, ((0, npad - N), (0, 0)))
    pos_p = jnp.pad(pos, ((0, npad - N), (0, 0)))
    bidx_p = jnp.pad(batch_index.astype(jnp.int32), (0, npad - N),
                     constant_values=G).reshape(nblocks, 1, BLK)
    ones = jnp.ones((npad, 1), jnp.float32)
    zeros3 = jnp.zeros((npad, 3), jnp.float32)
    zeros1 = jnp.zeros((npad, 1), jnp.float32)
    a_mat = jnp.concatenate([pos_p, zeros3, ones, zeros1], axis=1)
    b_mat = jnp.concatenate([zeros3, pos_p, zeros1, ones], axis=1)

    out = pl.pallas_call(
        functools.partial(_fused_kernel, nblocks=nblocks),
        grid=(nblocks,),
        in_specs=[
            pl.BlockSpec((BLK, H), lambda i: (i, 0)),
            pl.BlockSpec((BLK, 8), lambda i: (i, 0)),
            pl.BlockSpec((BLK, 8), lambda i: (i, 0)),
            pl.BlockSpec((1, 1, BLK), lambda i: (i, 0, 0)),
            pl.BlockSpec((H, H), lambda i: (0, 0)),
            pl.BlockSpec((1, H), lambda i: (0, 0)),
            pl.BlockSpec((H, H), lambda i: (0, 0)),
            pl.BlockSpec((1, H), lambda i: (0, 0)),
            pl.BlockSpec((H, 1), lambda i: (0, 0)),
        ],
        out_specs=pl.BlockSpec((G, 1), lambda i: (0, 0)),
        out_shape=jax.ShapeDtypeStruct((G, 1), jnp.float32),
        scratch_shapes=[pltpu.VMEM((G, 8), jnp.float32)],
        compiler_params=pltpu.CompilerParams(
            dimension_semantics=("arbitrary",),
        ),
    )(kemb_p, a_mat, b_mat, bidx_p,
      W1, b1.reshape(1, H), W2, b2.reshape(1, H), W_out)
    return out
